# baseline (speedup 1.0000x reference)
import functools
import math

import jax
import jax.numpy as jnp
from jax import lax
from jax.experimental import pallas as pl
from jax.experimental.pallas import tpu as pltpu
from jax.experimental.pallas import tpu_sc as plsc

N = 10000
E = 320000
D = 128
DH = 128

NC = 2
NS = 16
NW = NC * NS
EPW = E // NW
C = 80
NCH = EPW // C
LANES = D // 16

_F32 = jnp.float32


@functools.lru_cache(maxsize=None)
def _sc_pass(mul, acc, mat, summ, dual=False, nbuf=4):
    if acc and mul:
        nbuf = 2
    mesh = plsc.VectorSubcoreMesh(core_axis_name="c", subcore_axis_name="s",
                                  num_cores=NC, num_subcores=NS)
    outs = []
    if acc:
        outs.append(jax.ShapeDtypeStruct((NC * N, D), _F32))
    if mat:
        outs.append(jax.ShapeDtypeStruct((E, D), _F32))
        if dual:
            outs.append(jax.ShapeDtypeStruct((E, D), _F32))
    if summ:
        outs.append(jax.ShapeDtypeStruct((NW * 8, D), _F32))

    DMA = pltpu.SemaphoreType.DMA
    NB = nbuf
    scratch = ([pltpu.VMEM((C,), jnp.int32)] * NB + [DMA] * NB
               + [pltpu.VMEM((C, D), _F32)] * NB + [DMA] * NB)
    if dual:
        scratch += ([pltpu.VMEM((C,), jnp.int32)] * NB + [DMA] * NB
                    + [pltpu.VMEM((C, D), _F32)] * NB + [DMA] * NB)
    if mul:
        scratch += [pltpu.VMEM((C, D), _F32)] * NB + [DMA] * NB
    if acc:
        scratch += ([pltpu.VMEM((C,), jnp.int32)] * NB + [DMA] * NB
                    + [pltpu.VMEM_SHARED((N, D), _F32)])
    if mat:
        scratch += [DMA] * NB
        if dual:
            scratch += [DMA] * NB
    if summ:
        scratch += [pltpu.VMEM((8, D), _F32)]

    def body(*refs):
        it = iter(refs)

        def take(n):
            return [next(it) for _ in range(n)]

        tab = next(it)
        a_hbm = next(it) if mul else None
        idx_hbm = next(it)
        idx2_hbm = next(it) if dual else None
        dst_hbm = next(it) if acc else None
        zeros_hbm = next(it) if acc else None
        acc_out = next(it) if acc else None
        mat_out = next(it) if mat else None
        mat2_out = next(it) if (mat and dual) else None
        sum_out = next(it) if summ else None
        gidx = take(NB); isem = take(NB); rows = take(NB); gsem = take(NB)
        if dual:
            gidx2 = take(NB); i2sem = take(NB)
            rows2 = take(NB); g2sem = take(NB)
        if mul:
            a_v = take(NB); asem = take(NB)
        if acc:
            didx = take(NB); dsem = take(NB); acc_sh = next(it)
        if mat:
            msem = take(NB)
            if dual:
                m2sem = take(NB)
        if summ:
            vecbuf = next(it)

        cid = lax.axis_index("c")
        sid = lax.axis_index("s")
        wid = sid * NC + cid
        base = wid * EPW

        if acc:
            @pl.when(sid < NS - 1)
            def _():
                pltpu.sync_copy(zeros_hbm.at[pl.ds(sid * 624, 624)],
                                acc_sh.at[pl.ds(sid * 624, 624)])

            @pl.when(sid == NS - 1)
            def _():
                pltpu.sync_copy(zeros_hbm.at[pl.ds(9360, 640)],
                                acc_sh.at[pl.ds(9360, 640)])
            plsc.subcore_barrier()
        if summ:
            zv = jnp.zeros((16,), _F32)
            for r8 in range(8):
                for j in range(LANES):
                    vecbuf[r8, pl.ds(j * 16, 16)] = zv

        def i_desc(i, b):
            return pltpu.make_async_copy(
                idx_hbm.at[pl.ds(base + i * C, C)], gidx[b], isem[b])

        def i2_desc(i, b):
            return pltpu.make_async_copy(
                idx2_hbm.at[pl.ds(base + i * C, C)], gidx2[b], i2sem[b])

        def g_desc(b):
            return pltpu.make_async_copy(tab.at[gidx[b]], rows[b], gsem[b])

        def g2_desc(b):
            return pltpu.make_async_copy(tab.at[gidx2[b]], rows2[b], g2sem[b])

        def a_desc(i, b):
            return pltpu.make_async_copy(
                a_hbm.at[pl.ds(base + i * C, C)], a_v[b], asem[b])

        def d_desc(i, b):
            return pltpu.make_async_copy(
                dst_hbm.at[pl.ds(base + i * C, C)], didx[b], dsem[b])

        def loads(i, b):
            i_desc(i, b).start()
            if dual:
                i2_desc(i, b).start()
            if mul:
                a_desc(i, b).start()
            if acc:
                d_desc(i, b).start()

        def launch_gather(i, b, guard_mat):
            if mat and guard_mat:
                @pl.when(i >= NB)
                def _():
                    pltpu.make_async_copy(
                        rows[b], mat_out.at[pl.ds(base, C)], msem[b]).wait()
                    if dual:
                        pltpu.make_async_copy(
                            rows2[b], mat2_out.at[pl.ds(base, C)],
                            m2sem[b]).wait()
            i_desc(i, b).wait()
            g_desc(b).start()
            if dual:
                i2_desc(i, b).wait()
                g2_desc(b).start()

        def process(i, b):
            off = base + i * C
            g_desc(b).wait()
            if dual:
                g2_desc(b).wait()
            if mul:
                a_desc(i, b).wait()
            if summ:
                def srow(r4, carry):
                    r = r4 * 4
                    for u in range(4):
                        new = []
                        for j in range(LANES):
                            s = pl.ds(j * 16, 16)
                            v = rows[b][r + u, s]
                            if mul:
                                v = v * a_v[b][r + u, s]
                            new.append(carry[j] + v)
                        carry = tuple(new)
                    return carry
                carry = lax.fori_loop(
                    0, C // 4, srow,
                    tuple(jnp.zeros((16,), _F32) for _ in range(LANES)))
                for j in range(LANES):
                    s = pl.ds(j * 16, 16)
                    vecbuf[0, s] = vecbuf[0, s] + carry[j]
            elif mul:
                def mrow(r4, x):
                    r = r4 * 4
                    for u in range(4):
                        for j in range(LANES):
                            s = pl.ds(j * 16, 16)
                            rows[b][r + u, s] = (rows[b][r + u, s]
                                                 * a_v[b][r + u, s])
                    return x
                lax.fori_loop(0, C // 4, mrow, 0)
            if mat:
                pltpu.async_copy(rows[b], mat_out.at[pl.ds(off, C)], msem[b])
                if dual:
                    pltpu.async_copy(rows2[b], mat2_out.at[pl.ds(off, C)],
                                     m2sem[b])
            if acc:
                d_desc(i, b).wait()
                pltpu.sync_copy(rows[b], acc_sh.at[didx[b]], add=True)

        for i in range(NB):
            loads(i, i)
        for i in range(NB - 1):
            launch_gather(i, i, False)

        def iter_body(g, x):
            ph = lax.rem(g, NB)
            for b in range(NB):
                @pl.when(ph == b)
                def _(b=b):
                    process(g, b)

                @pl.when(jnp.logical_and(ph == b, g + NB - 1 < NCH))
                def _(b=b):
                    launch_gather(g + NB - 1, (b + NB - 1) % NB, True)

                @pl.when(jnp.logical_and(ph == b, g + NB < NCH))
                def _(b=b):
                    loads(g + NB, b)
            return x
        lax.fori_loop(0, NCH, iter_body, 0)

        if mat:
            for b in range(NB):
                pltpu.make_async_copy(rows[b], mat_out.at[pl.ds(base, C)],
                                      msem[b]).wait()
                if dual:
                    pltpu.make_async_copy(rows2[b],
                                          mat2_out.at[pl.ds(base, C)],
                                          m2sem[b]).wait()
        if acc:
            plsc.subcore_barrier()

            @pl.when(sid < NS - 1)
            def _():
                pltpu.sync_copy(acc_sh.at[pl.ds(sid * 624, 624)],
                                acc_out.at[pl.ds(cid * N + sid * 624, 624)])

            @pl.when(sid == NS - 1)
            def _():
                pltpu.sync_copy(acc_sh.at[pl.ds(9360, 640)],
                                acc_out.at[pl.ds(cid * N + 9360, 640)])
        if summ:
            pltpu.sync_copy(vecbuf, sum_out.at[pl.ds(wid * 8, 8)])

    return pl.kernel(body, out_type=tuple(outs), mesh=mesh,
                     scratch_types=scratch)


_NB = 1000


def _tc_linear(p0, p1, W, b, relu):
    def lin_body(p0_ref, p1_ref, w_ref, b_ref, o_ref):
        s = p0_ref[...] + p1_ref[...]
        h = jnp.dot(s, w_ref[...], preferred_element_type=_F32) + b_ref[...]
        o_ref[...] = jnp.maximum(h, 0.0) if relu else h

    return pl.pallas_call(
        lin_body,
        grid=(N // _NB,),
        in_specs=[
            pl.BlockSpec((_NB, D), lambda i: (i, 0)),
            pl.BlockSpec((_NB, D), lambda i: (i, 0)),
            pl.BlockSpec((D, D), lambda i: (0, 0)),
            pl.BlockSpec((1, D), lambda i: (0, 0)),
        ],
        out_specs=pl.BlockSpec((_NB, D), lambda i: (i, 0)),
        out_shape=jax.ShapeDtypeStruct((N, D), _F32),
    )(p0, p1, W, b)


_EB = 1600


def _tc_edge_mlp(zs, zd, ef, er, Wt, Wb, ball):
    def edge_body(zs_ref, zd_ref, ef_ref, er_ref, wt_ref, wb_ref,
                  b_ref, af_ref, ar_ref, s1_ref, s2_ref):
        i = pl.program_id(0)
        h = (jnp.dot(zs_ref[...], wt_ref[...], preferred_element_type=_F32)
             + jnp.dot(zd_ref[...], wb_ref[...], preferred_element_type=_F32)
             + b_ref[...])
        af = h[:, 0:D] + jnp.exp(h[:, D:2 * D]) * ef_ref[...]
        ar = h[:, 2 * D:3 * D] + jnp.exp(h[:, 3 * D:4 * D]) * er_ref[...]
        af_ref[...] = af
        ar_ref[...] = ar

        @pl.when(i == 0)
        def _():
            s1_ref[...] = jnp.zeros_like(s1_ref)
            s2_ref[...] = jnp.zeros_like(s2_ref)

        s1_ref[...] += jnp.sum((af - 1.0) ** 2, axis=0, keepdims=True)
        s2_ref[...] += jnp.sum((ar - 1.0) ** 2, axis=0, keepdims=True)

    return pl.pallas_call(
        edge_body,
        grid=(E // _EB,),
        in_specs=[
            pl.BlockSpec((_EB, D), lambda i: (i, 0)),
            pl.BlockSpec((_EB, D), lambda i: (i, 0)),
            pl.BlockSpec((_EB, D), lambda i: (i, 0)),
            pl.BlockSpec((_EB, D), lambda i: (i, 0)),
            pl.BlockSpec((D, 4 * D), lambda i: (0, 0)),
            pl.BlockSpec((D, 4 * D), lambda i: (0, 0)),
            pl.BlockSpec((1, 4 * D), lambda i: (0, 0)),
        ],
        out_specs=[
            pl.BlockSpec((_EB, D), lambda i: (i, 0)),
            pl.BlockSpec((_EB, D), lambda i: (i, 0)),
            pl.BlockSpec((1, D), lambda i: (0, 0)),
            pl.BlockSpec((1, D), lambda i: (0, 0)),
        ],
        out_shape=[
            jax.ShapeDtypeStruct((E, D), _F32),
            jax.ShapeDtypeStruct((E, D), _F32),
            jax.ShapeDtypeStruct((1, D), _F32),
            jax.ShapeDtypeStruct((1, D), _F32),
        ],
    )(zs, zd, ef, er, Wt, Wb, ball)


def _tc_final(parts, s1, s2, Wg2, bg2, Wd1, bd1, Wd2, bd2):
    log2pi = math.log(2.0 * math.pi)

    def final_body(p_ref, s1_ref, s2_ref, wg2_ref, bg2_ref, wd1_ref, bd1_ref,
                   wd2_ref, bd2_ref, o_ref, nll_ref):
        s = jnp.sum(p_ref[...], axis=0, keepdims=True)
        o = jnp.dot(s, wg2_ref[...], preferred_element_type=_F32) \
            + float(N) * bg2_ref[...]
        o = jnp.maximum(o, 0.0)
        o = jnp.dot(o, wd1_ref[...], preferred_element_type=_F32) + bd1_ref[...]
        o = jnp.maximum(o, 0.0)
        o = jnp.dot(o, wd2_ref[...], preferred_element_type=_F32) + bd2_ref[...]
        o_ref[...] = o
        tot = jnp.sum(s1_ref[...]) + jnp.sum(s2_ref[...])
        nll = log2pi + 0.5 * tot / float(E * D)
        nll_ref[...] = jnp.full((1, 1), nll, _F32)

    return pl.pallas_call(
        final_body,
        out_shape=[jax.ShapeDtypeStruct((1, D), _F32),
                   jax.ShapeDtypeStruct((1, 1), _F32)],
    )(parts, s1, s2, Wg2, bg2, Wd1, bd1, Wd2, bd2)


def kernel(x, edge_index, W_enc0, b_enc0, W_enc1, b_enc1, W_mu, b_mu, W_ls,
           b_ls, W_muf, b_muf, W_lsf, b_lsf, W_g0, b_g0, W_g1, b_g1, W_g2,
           b_g2, W_d1, b_d1, W_d2, b_d2, eps_first, eps_rest):
    src = edge_index[0]
    dst = edge_index[1]
    zeros = jnp.zeros((N, D), _F32)

    Wt = jnp.concatenate([W_muf[:DH], W_lsf[:DH], W_mu[:DH], W_ls[:DH]], axis=1)
    Wb = jnp.concatenate([W_muf[DH:], W_lsf[DH:], W_mu[DH:], W_ls[DH:]], axis=1)
    ball = jnp.concatenate([b_muf, b_lsf, b_mu, b_ls]).reshape(1, 4 * D)

    (agg0,) = _sc_pass(False, True, False, False)(x, src, dst, zeros)
    z1 = _tc_linear(agg0[:N], agg0[N:], W_enc0, b_enc0.reshape(1, D), True)
    (agg1,) = _sc_pass(False, True, False, False)(z1, src, dst, zeros)
    z2 = _tc_linear(agg1[:N], agg1[N:], W_enc1, b_enc1.reshape(1, D), True)
    z2src, z2dst = _sc_pass(False, False, True, False, dual=True)(z2, src, dst)
    a_first, a_rest, s1, s2 = _tc_edge_mlp(z2src, z2dst, eps_first, eps_rest,
                                           Wt, Wb, ball)
    (agg2,) = _sc_pass(True, True, False, False)(x, a_first, src, dst, zeros)
    h1 = _tc_linear(agg2[:N], agg2[N:], W_g0, b_g0.reshape(1, D), True)
    (agg3,) = _sc_pass(True, True, False, False)(h1, a_rest, src, dst, zeros)
    h2 = _tc_linear(agg3[:N], agg3[N:], W_g1, b_g1.reshape(1, D), True)
    (parts,) = _sc_pass(True, False, False, True)(h2, a_rest, src)
    out, nll = _tc_final(parts, s1, s2, W_g2, b_g2.reshape(1, D),
                         W_d1, b_d1.reshape(1, D), W_d2, b_d2.reshape(1, D))
    return (out, nll.reshape(()))


# speedup vs baseline: 1.0603x; 1.0603x over previous
import functools
import math

import jax
import jax.numpy as jnp
from jax import lax
from jax.experimental import pallas as pl
from jax.experimental.pallas import tpu as pltpu
from jax.experimental.pallas import tpu_sc as plsc

N = 10000
E = 320000
D = 128
DH = 128

NC = 2
NS = 16
NW = NC * NS
EPW = E // NW
C = 80
NCH = EPW // C
LANES = D // 16

_F32 = jnp.float32


@functools.lru_cache(maxsize=None)
def _sc_pass(mul, acc, mat, summ, dual=False, nbuf=4):
    if acc and mul:
        nbuf = 2
    mesh = plsc.VectorSubcoreMesh(core_axis_name="c", subcore_axis_name="s",
                                  num_cores=NC, num_subcores=NS)
    outs = []
    if acc:
        outs.append(jax.ShapeDtypeStruct((NC * N, D), _F32))
    if mat:
        outs.append(jax.ShapeDtypeStruct((E, D), _F32))
        if dual:
            outs.append(jax.ShapeDtypeStruct((E, D), _F32))
    if summ:
        outs.append(jax.ShapeDtypeStruct((NW * 8, D), _F32))

    DMA = pltpu.SemaphoreType.DMA
    NB = nbuf
    scratch = ([pltpu.VMEM((C,), jnp.int32)] * NB + [DMA] * NB
               + [pltpu.VMEM((C, D), _F32)] * NB + [DMA] * NB)
    if dual:
        scratch += ([pltpu.VMEM((C,), jnp.int32)] * NB + [DMA] * NB
                    + [pltpu.VMEM((C, D), _F32)] * NB + [DMA] * NB)
    if mul:
        scratch += [pltpu.VMEM((C, D), _F32)] * NB + [DMA] * NB
    if acc:
        scratch += ([pltpu.VMEM((C,), jnp.int32)] * NB + [DMA] * NB
                    + [pltpu.VMEM_SHARED((N, D), _F32)])
    if mat:
        scratch += [DMA] * NB
        if dual:
            scratch += [DMA] * NB
    if summ:
        scratch += [pltpu.VMEM((8, D), _F32)]

    def body(*refs):
        it = iter(refs)

        def take(n):
            return [next(it) for _ in range(n)]

        tab = next(it)
        a_hbm = next(it) if mul else None
        idx_hbm = next(it)
        idx2_hbm = next(it) if dual else None
        dst_hbm = next(it) if acc else None
        zeros_hbm = next(it) if acc else None
        acc_out = next(it) if acc else None
        mat_out = next(it) if mat else None
        mat2_out = next(it) if (mat and dual) else None
        sum_out = next(it) if summ else None
        gidx = take(NB); isem = take(NB); rows = take(NB); gsem = take(NB)
        if dual:
            gidx2 = take(NB); i2sem = take(NB)
            rows2 = take(NB); g2sem = take(NB)
        if mul:
            a_v = take(NB); asem = take(NB)
        if acc:
            didx = take(NB); dsem = take(NB); acc_sh = next(it)
        if mat:
            msem = take(NB)
            if dual:
                m2sem = take(NB)
        if summ:
            vecbuf = next(it)

        cid = lax.axis_index("c")
        sid = lax.axis_index("s")
        wid = sid * NC + cid
        base = wid * EPW

        if acc:
            @pl.when(sid < NS - 1)
            def _():
                pltpu.sync_copy(zeros_hbm.at[pl.ds(sid * 624, 624)],
                                acc_sh.at[pl.ds(sid * 624, 624)])

            @pl.when(sid == NS - 1)
            def _():
                pltpu.sync_copy(zeros_hbm.at[pl.ds(9360, 640)],
                                acc_sh.at[pl.ds(9360, 640)])
            plsc.subcore_barrier()
        if summ:
            zv = jnp.zeros((16,), _F32)
            for r8 in range(8):
                for j in range(LANES):
                    vecbuf[r8, pl.ds(j * 16, 16)] = zv

        def i_desc(i, b):
            return pltpu.make_async_copy(
                idx_hbm.at[pl.ds(base + i * C, C)], gidx[b], isem[b])

        def i2_desc(i, b):
            return pltpu.make_async_copy(
                idx2_hbm.at[pl.ds(base + i * C, C)], gidx2[b], i2sem[b])

        def g_desc(b):
            return pltpu.make_async_copy(tab.at[gidx[b]], rows[b], gsem[b])

        def g2_desc(b):
            return pltpu.make_async_copy(tab.at[gidx2[b]], rows2[b], g2sem[b])

        def a_desc(i, b):
            return pltpu.make_async_copy(
                a_hbm.at[pl.ds(base + i * C, C)], a_v[b], asem[b])

        def d_desc(i, b):
            return pltpu.make_async_copy(
                dst_hbm.at[pl.ds(base + i * C, C)], didx[b], dsem[b])

        def loads(i, b):
            i_desc(i, b).start()
            if dual:
                i2_desc(i, b).start()
            if mul:
                a_desc(i, b).start()
            if acc:
                d_desc(i, b).start()

        def launch_gather(i, b, guard_mat):
            if mat and guard_mat:
                @pl.when(i >= NB)
                def _():
                    pltpu.make_async_copy(
                        rows[b], mat_out.at[pl.ds(base, C)], msem[b]).wait()
                    if dual:
                        pltpu.make_async_copy(
                            rows2[b], mat2_out.at[pl.ds(base, C)],
                            m2sem[b]).wait()
            i_desc(i, b).wait()
            g_desc(b).start()
            if dual:
                i2_desc(i, b).wait()
                g2_desc(b).start()

        def process(i, b):
            off = base + i * C
            g_desc(b).wait()
            if dual:
                g2_desc(b).wait()
            if mul:
                a_desc(i, b).wait()
            if summ:
                def srow(r4, carry):
                    r = r4 * 4
                    for u in range(4):
                        new = []
                        for j in range(LANES):
                            s = pl.ds(j * 16, 16)
                            v = rows[b][r + u, s]
                            if mul:
                                v = v * a_v[b][r + u, s]
                            new.append(carry[j] + v)
                        carry = tuple(new)
                    return carry
                carry = lax.fori_loop(
                    0, C // 4, srow,
                    tuple(jnp.zeros((16,), _F32) for _ in range(LANES)))
                for j in range(LANES):
                    s = pl.ds(j * 16, 16)
                    vecbuf[0, s] = vecbuf[0, s] + carry[j]
            elif mul:
                def mrow(r4, x):
                    r = r4 * 4
                    for u in range(4):
                        for j in range(LANES):
                            s = pl.ds(j * 16, 16)
                            rows[b][r + u, s] = (rows[b][r + u, s]
                                                 * a_v[b][r + u, s])
                    return x
                lax.fori_loop(0, C // 4, mrow, 0)
            if mat:
                pltpu.async_copy(rows[b], mat_out.at[pl.ds(off, C)], msem[b])
                if dual:
                    pltpu.async_copy(rows2[b], mat2_out.at[pl.ds(off, C)],
                                     m2sem[b])
            if acc:
                d_desc(i, b).wait()
                pltpu.sync_copy(rows[b], acc_sh.at[didx[b]], add=True)

        for i in range(NB):
            loads(i, i)
        for i in range(NB - 1):
            launch_gather(i, i, False)

        def iter_body(g, x):
            ph = lax.rem(g, NB)
            for b in range(NB):
                @pl.when(jnp.logical_and(ph == b, g + NB - 1 < NCH))
                def _(b=b):
                    launch_gather(g + NB - 1, (b + NB - 1) % NB, True)

                @pl.when(ph == b)
                def _(b=b):
                    process(g, b)

                @pl.when(jnp.logical_and(ph == b, g + NB < NCH))
                def _(b=b):
                    loads(g + NB, b)
            return x
        lax.fori_loop(0, NCH, iter_body, 0)

        if mat:
            for b in range(NB):
                pltpu.make_async_copy(rows[b], mat_out.at[pl.ds(base, C)],
                                      msem[b]).wait()
                if dual:
                    pltpu.make_async_copy(rows2[b],
                                          mat2_out.at[pl.ds(base, C)],
                                          m2sem[b]).wait()
        if acc:
            plsc.subcore_barrier()

            @pl.when(sid < NS - 1)
            def _():
                pltpu.sync_copy(acc_sh.at[pl.ds(sid * 624, 624)],
                                acc_out.at[pl.ds(cid * N + sid * 624, 624)])

            @pl.when(sid == NS - 1)
            def _():
                pltpu.sync_copy(acc_sh.at[pl.ds(9360, 640)],
                                acc_out.at[pl.ds(cid * N + 9360, 640)])
        if summ:
            pltpu.sync_copy(vecbuf, sum_out.at[pl.ds(wid * 8, 8)])

    return pl.kernel(body, out_type=tuple(outs), mesh=mesh,
                     scratch_types=scratch)


_NB = 1000


def _tc_linear(p0, p1, W, b, relu):
    def lin_body(p0_ref, p1_ref, w_ref, b_ref, o_ref):
        s = p0_ref[...] + p1_ref[...]
        h = jnp.dot(s, w_ref[...], preferred_element_type=_F32) + b_ref[...]
        o_ref[...] = jnp.maximum(h, 0.0) if relu else h

    return pl.pallas_call(
        lin_body,
        grid=(N // _NB,),
        in_specs=[
            pl.BlockSpec((_NB, D), lambda i: (i, 0)),
            pl.BlockSpec((_NB, D), lambda i: (i, 0)),
            pl.BlockSpec((D, D), lambda i: (0, 0)),
            pl.BlockSpec((1, D), lambda i: (0, 0)),
        ],
        out_specs=pl.BlockSpec((_NB, D), lambda i: (i, 0)),
        out_shape=jax.ShapeDtypeStruct((N, D), _F32),
    )(p0, p1, W, b)


_EB = 1600


def _tc_edge_mlp(zs, zd, ef, er, Wt, Wb, ball):
    def edge_body(zs_ref, zd_ref, ef_ref, er_ref, wt_ref, wb_ref,
                  b_ref, af_ref, ar_ref, s1_ref, s2_ref):
        i = pl.program_id(0)
        h = (jnp.dot(zs_ref[...], wt_ref[...], preferred_element_type=_F32)
             + jnp.dot(zd_ref[...], wb_ref[...], preferred_element_type=_F32)
             + b_ref[...])
        af = h[:, 0:D] + jnp.exp(h[:, D:2 * D]) * ef_ref[...]
        ar = h[:, 2 * D:3 * D] + jnp.exp(h[:, 3 * D:4 * D]) * er_ref[...]
        af_ref[...] = af
        ar_ref[...] = ar

        @pl.when(i == 0)
        def _():
            s1_ref[...] = jnp.zeros_like(s1_ref)
            s2_ref[...] = jnp.zeros_like(s2_ref)

        s1_ref[...] += jnp.sum((af - 1.0) ** 2, axis=0, keepdims=True)
        s2_ref[...] += jnp.sum((ar - 1.0) ** 2, axis=0, keepdims=True)

    return pl.pallas_call(
        edge_body,
        grid=(E // _EB,),
        in_specs=[
            pl.BlockSpec((_EB, D), lambda i: (i, 0)),
            pl.BlockSpec((_EB, D), lambda i: (i, 0)),
            pl.BlockSpec((_EB, D), lambda i: (i, 0)),
            pl.BlockSpec((_EB, D), lambda i: (i, 0)),
            pl.BlockSpec((D, 4 * D), lambda i: (0, 0)),
            pl.BlockSpec((D, 4 * D), lambda i: (0, 0)),
            pl.BlockSpec((1, 4 * D), lambda i: (0, 0)),
        ],
        out_specs=[
            pl.BlockSpec((_EB, D), lambda i: (i, 0)),
            pl.BlockSpec((_EB, D), lambda i: (i, 0)),
            pl.BlockSpec((1, D), lambda i: (0, 0)),
            pl.BlockSpec((1, D), lambda i: (0, 0)),
        ],
        out_shape=[
            jax.ShapeDtypeStruct((E, D), _F32),
            jax.ShapeDtypeStruct((E, D), _F32),
            jax.ShapeDtypeStruct((1, D), _F32),
            jax.ShapeDtypeStruct((1, D), _F32),
        ],
    )(zs, zd, ef, er, Wt, Wb, ball)


def _tc_final(parts, s1, s2, Wg2, bg2, Wd1, bd1, Wd2, bd2):
    log2pi = math.log(2.0 * math.pi)

    def final_body(p_ref, s1_ref, s2_ref, wg2_ref, bg2_ref, wd1_ref, bd1_ref,
                   wd2_ref, bd2_ref, o_ref, nll_ref):
        s = jnp.sum(p_ref[...], axis=0, keepdims=True)
        o = jnp.dot(s, wg2_ref[...], preferred_element_type=_F32) \
            + float(N) * bg2_ref[...]
        o = jnp.maximum(o, 0.0)
        o = jnp.dot(o, wd1_ref[...], preferred_element_type=_F32) + bd1_ref[...]
        o = jnp.maximum(o, 0.0)
        o = jnp.dot(o, wd2_ref[...], preferred_element_type=_F32) + bd2_ref[...]
        o_ref[...] = o
        tot = jnp.sum(s1_ref[...]) + jnp.sum(s2_ref[...])
        nll = log2pi + 0.5 * tot / float(E * D)
        nll_ref[...] = jnp.full((1, 1), nll, _F32)

    return pl.pallas_call(
        final_body,
        out_shape=[jax.ShapeDtypeStruct((1, D), _F32),
                   jax.ShapeDtypeStruct((1, 1), _F32)],
    )(parts, s1, s2, Wg2, bg2, Wd1, bd1, Wd2, bd2)


def kernel(x, edge_index, W_enc0, b_enc0, W_enc1, b_enc1, W_mu, b_mu, W_ls,
           b_ls, W_muf, b_muf, W_lsf, b_lsf, W_g0, b_g0, W_g1, b_g1, W_g2,
           b_g2, W_d1, b_d1, W_d2, b_d2, eps_first, eps_rest):
    src = edge_index[0]
    dst = edge_index[1]
    zeros = jnp.zeros((N, D), _F32)

    Wt = jnp.concatenate([W_muf[:DH], W_lsf[:DH], W_mu[:DH], W_ls[:DH]], axis=1)
    Wb = jnp.concatenate([W_muf[DH:], W_lsf[DH:], W_mu[DH:], W_ls[DH:]], axis=1)
    ball = jnp.concatenate([b_muf, b_lsf, b_mu, b_ls]).reshape(1, 4 * D)

    (agg0,) = _sc_pass(False, True, False, False)(x, src, dst, zeros)
    z1 = _tc_linear(agg0[:N], agg0[N:], W_enc0, b_enc0.reshape(1, D), True)
    (agg1,) = _sc_pass(False, True, False, False)(z1, src, dst, zeros)
    z2 = _tc_linear(agg1[:N], agg1[N:], W_enc1, b_enc1.reshape(1, D), True)
    z2src, z2dst = _sc_pass(False, False, True, False, dual=True)(z2, src, dst)
    a_first, a_rest, s1, s2 = _tc_edge_mlp(z2src, z2dst, eps_first, eps_rest,
                                           Wt, Wb, ball)
    (agg2,) = _sc_pass(True, True, False, False)(x, a_first, src, dst, zeros)
    h1 = _tc_linear(agg2[:N], agg2[N:], W_g0, b_g0.reshape(1, D), True)
    (agg3,) = _sc_pass(True, True, False, False)(h1, a_rest, src, dst, zeros)
    h2 = _tc_linear(agg3[:N], agg3[N:], W_g1, b_g1.reshape(1, D), True)
    (parts,) = _sc_pass(True, False, False, True)(h2, a_rest, src)
    out, nll = _tc_final(parts, s1, s2, W_g2, b_g2.reshape(1, D),
                         W_d1, b_d1.reshape(1, D), W_d2, b_d2.reshape(1, D))
    return (out, nll.reshape(()))
